# Initial kernel scaffold; baseline (speedup 1.0000x reference)
import functools

import jax
import jax.numpy as jnp
from jax import lax
from jax.experimental import pallas as pl
from jax.experimental.pallas import tpu as pltpu
from jax.experimental.pallas import tpu_sc as plsc

_NC = 2
_NS = 16
_NW = _NC * _NS
_L = 16
_K = 80


@functools.lru_cache(maxsize=None)
def _sc_agg(n, d, ch, with_deg):
    mesh = plsc.VectorSubcoreMesh(core_axis_name="c", subcore_axis_name="s")
    out_type = [jax.ShapeDtypeStruct((_NC, n, d), jnp.float32)]
    if with_deg:
        out_type.append(jax.ShapeDtypeStruct((_NC, n, _L), jnp.float32))

    scratch = [
        pltpu.VMEM((ch, _K), jnp.int32),
        pltpu.VMEM((ch, _K), jnp.int32),
        pltpu.VMEM((_K, d), jnp.float32),
        pltpu.VMEM((_K, d), jnp.float32),
        pltpu.VMEM((_K, _L), jnp.float32),
        pltpu.VMEM_SHARED((n, d), jnp.float32),
        pltpu.VMEM_SHARED((n, _L), jnp.float32),
        pltpu.SemaphoreType.DMA,
        pltpu.SemaphoreType.DMA,
    ]

    rows_per_tile = n // _NS

    def body(table, src, dst, zrows, zdeg, *rest):
        if with_deg:
            out_p, out_deg = rest[0], rest[1]
            rest = rest[2:]
        else:
            out_p = rest[0]
            rest = rest[1:]
        srcb, dstb, rows0, rows1, ones, acc, dacc, sem0, sem1 = rest

        cid = lax.axis_index("c")
        sid = lax.axis_index("s")
        wid = cid * _NS + sid

        pltpu.sync_copy(src.at[wid], srcb)
        pltpu.sync_copy(dst.at[wid], dstb)

        lo = sid * rows_per_tile
        pltpu.sync_copy(zrows.at[pl.ds(lo, rows_per_tile)],
                        acc.at[pl.ds(lo, rows_per_tile)])
        if with_deg:
            pltpu.sync_copy(zdeg.at[pl.ds(lo, rows_per_tile)],
                            dacc.at[pl.ds(lo, rows_per_tile)])
            for j in range(_K):
                ones[j, :] = jnp.ones((_L,), jnp.float32)
        plsc.subcore_barrier()

        def start(ci, rows, sem):
            pltpu.make_async_copy(table.at[srcb.at[ci]], rows, sem).start()

        def finish(ci, rows, sem):
            pltpu.make_async_copy(table.at[srcb.at[ci]], rows, sem).wait()
            pltpu.sync_copy(rows, acc.at[dstb.at[ci]], add=True)
            if with_deg:
                pltpu.sync_copy(ones, dacc.at[dstb.at[ci]], add=True)

        half = (ch - 1) // 2
        start(0, rows0, sem0)
        start(1, rows1, sem1)

        def loop(gp, carry):
            c0 = 2 * gp
            finish(c0, rows0, sem0)

            @pl.when(c0 + 2 < ch)
            def _():
                start(c0 + 2, rows0, sem0)

            finish(c0 + 1, rows1, sem1)

            @pl.when(c0 + 3 < ch)
            def _():
                start(c0 + 3, rows1, sem1)

            return carry

        lax.fori_loop(0, half, loop, 0)
        finish(ch - 1, rows0, sem0)

        plsc.subcore_barrier()
        pltpu.sync_copy(acc.at[pl.ds(lo, rows_per_tile)],
                        out_p.at[cid, pl.ds(lo, rows_per_tile)])
        if with_deg:
            pltpu.sync_copy(dacc.at[pl.ds(lo, rows_per_tile)],
                            out_deg.at[cid, pl.ds(lo, rows_per_tile)])

    return pl.kernel(body, mesh=mesh, out_type=out_type,
                     scratch_types=scratch)


@functools.lru_cache(maxsize=None)
def _tc_combine(n, d, h, relu, block_rows):
    def body(x_ref, p_ref, dg_ref, ws_ref, wn_ref, b_ref, o_ref):
        deg = dg_ref[0, :, 0:1] + dg_ref[1, :, 0:1]
        hn = (p_ref[0] + p_ref[1]) / jnp.maximum(deg, 1.0)
        acc = jnp.dot(x_ref[...], ws_ref[...],
                      preferred_element_type=jnp.float32)
        acc += jnp.dot(hn, wn_ref[...], preferred_element_type=jnp.float32)
        acc += b_ref[...]
        o_ref[...] = jnp.maximum(acc, 0.0) if relu else acc

    grid = n // block_rows
    return pl.pallas_call(
        body,
        grid=(grid,),
        in_specs=[
            pl.BlockSpec((block_rows, d), lambda i: (i, 0)),
            pl.BlockSpec((2, block_rows, d), lambda i: (0, i, 0)),
            pl.BlockSpec((2, block_rows, _L), lambda i: (0, i, 0)),
            pl.BlockSpec((d, h), lambda i: (0, 0)),
            pl.BlockSpec((d, h), lambda i: (0, 0)),
            pl.BlockSpec((1, h), lambda i: (0, 0)),
        ],
        out_specs=pl.BlockSpec((block_rows, h), lambda i: (i, 0)),
        out_shape=jax.ShapeDtypeStruct((n, h), jnp.float32),
    )


def kernel(x, edge_index, W1_self, W1_neigh, b1, W2_self, W2_neigh, b2):
    n, d = x.shape
    e = edge_index.shape[1]
    epw = e // _NW
    ch = epw // _K

    src = edge_index[0].reshape(_NW, ch, _K)
    dst = edge_index[1].reshape(_NW, ch, _K)
    zrows = jnp.zeros((n, d), jnp.float32)
    zdeg = jnp.zeros((n, _L), jnp.float32)

    p1, degt = _sc_agg(n, d, ch, True)(x, src, dst, zrows, zdeg)
    h1 = _tc_combine(n, d, W1_self.shape[1], True, 1000)(
        x, p1, degt, W1_self, W1_neigh, b1.reshape(1, -1))
    (p2,) = _sc_agg(n, d, ch, False)(h1, src, dst, zrows, zdeg)
    h2 = _tc_combine(n, d, W2_self.shape[1], False, 1000)(
        h1, p2, degt, W2_self, W2_neigh, b2.reshape(1, -1))
    return h2


# speedup vs baseline: 9.1303x; 9.1303x over previous
import functools

import jax
import jax.numpy as jnp
from jax import lax
from jax.experimental import pallas as pl
from jax.experimental.pallas import tpu as pltpu
from jax.experimental.pallas import tpu_sc as plsc

_NC = 2
_NS = 16
_L = 16
_K = 80


@functools.lru_cache(maxsize=None)
def _sc_agg(n, dh, ch, with_deg):
    mesh = plsc.VectorSubcoreMesh(core_axis_name="c", subcore_axis_name="s")
    out_type = [jax.ShapeDtypeStruct((_NC, n, dh), jnp.float32)]
    if with_deg:
        out_type.append(jax.ShapeDtypeStruct((n, _L), jnp.float32))

    scratch = [
        pltpu.VMEM((ch, _K), jnp.int32),
        pltpu.VMEM((ch, _K), jnp.int32),
        pltpu.VMEM((_K, dh), jnp.float32),
        pltpu.VMEM((_K, dh), jnp.float32),
        pltpu.VMEM((_K, _L), jnp.float32),
        pltpu.VMEM_SHARED((n, dh), jnp.float32),
        pltpu.VMEM_SHARED((n, _L), jnp.float32),
        pltpu.SemaphoreType.DMA,
        pltpu.SemaphoreType.DMA,
    ]

    stride = (n // 8 // _NS) * 8
    window = n - (_NS - 1) * stride

    def body(table, src, dst, zrows, zdeg, *rest):
        if with_deg:
            out_p, out_deg = rest[0], rest[1]
            rest = rest[2:]
        else:
            out_p = rest[0]
            rest = rest[1:]
        srcb, dstb, rows0, rows1, ones, acc, dacc, sem0, sem1 = rest

        cid = lax.axis_index("c")
        sid = lax.axis_index("s")
        half_tab = table.at[cid]

        pltpu.sync_copy(src.at[sid], srcb)
        pltpu.sync_copy(dst.at[sid], dstb)

        lo = sid * stride
        pltpu.sync_copy(zrows.at[pl.ds(lo, window)],
                        acc.at[pl.ds(lo, window)])
        if with_deg:
            pltpu.sync_copy(zdeg.at[pl.ds(lo, window)],
                            dacc.at[pl.ds(lo, window)])
            for j in range(_K):
                ones[j, :] = jnp.ones((_L,), jnp.float32)
        plsc.subcore_barrier()

        def start(ci, rows, sem):
            pltpu.make_async_copy(half_tab.at[srcb.at[ci]], rows, sem).start()

        def finish(ci, rows, sem):
            pltpu.make_async_copy(half_tab.at[srcb.at[ci]], rows, sem).wait()
            pltpu.sync_copy(rows, acc.at[dstb.at[ci]], add=True)
            if with_deg:
                @pl.when(cid == 0)
                def _():
                    pltpu.sync_copy(ones, dacc.at[dstb.at[ci]], add=True)

        start(0, rows0, sem0)
        start(1, rows1, sem1)

        def loop(gp, carry):
            c0 = 2 * gp
            finish(c0, rows0, sem0)

            @pl.when(c0 + 2 < ch)
            def _():
                start(c0 + 2, rows0, sem0)

            finish(c0 + 1, rows1, sem1)

            @pl.when(c0 + 3 < ch)
            def _():
                start(c0 + 3, rows1, sem1)

            return carry

        lax.fori_loop(0, ch // 2, loop, 0)

        plsc.subcore_barrier()
        pltpu.sync_copy(acc.at[pl.ds(lo, window)],
                        out_p.at[cid, pl.ds(lo, window)])
        if with_deg:
            @pl.when(cid == 0)
            def _():
                pltpu.sync_copy(dacc.at[pl.ds(lo, window)],
                                out_deg.at[pl.ds(lo, window)])

    return pl.kernel(body, mesh=mesh, out_type=out_type,
                     scratch_types=scratch,
                     compiler_params=pltpu.CompilerParams(
                         use_tc_tiling_on_sc=False))


@functools.lru_cache(maxsize=None)
def _tc_combine(n, d, h, relu, stacked_out, block_rows):
    dh = d // 2

    def body(x_ref, p_ref, dg_ref, ws_ref, wn_ref, b_ref, o_ref):
        inv = 1.0 / jnp.maximum(dg_ref[:, 0:1], 1.0)
        acc = jnp.dot(x_ref[...], ws_ref[...],
                      preferred_element_type=jnp.float32)
        acc += jnp.dot(p_ref[0] * inv, wn_ref[0:dh, :],
                       preferred_element_type=jnp.float32)
        acc += jnp.dot(p_ref[1] * inv, wn_ref[dh:, :],
                       preferred_element_type=jnp.float32)
        acc += b_ref[...]
        acc = jnp.maximum(acc, 0.0) if relu else acc
        if stacked_out:
            o_ref[0] = acc[:, : h // 2]
            o_ref[1] = acc[:, h // 2:]
        else:
            o_ref[...] = acc

    if stacked_out:
        out_spec = pl.BlockSpec((2, block_rows, h // 2), lambda i: (0, i, 0))
        out_shape = jax.ShapeDtypeStruct((2, n, h // 2), jnp.float32)
    else:
        out_spec = pl.BlockSpec((block_rows, h), lambda i: (i, 0))
        out_shape = jax.ShapeDtypeStruct((n, h), jnp.float32)

    return pl.pallas_call(
        body,
        grid=(n // block_rows,),
        in_specs=[
            pl.BlockSpec((block_rows, d), lambda i: (i, 0)),
            pl.BlockSpec((2, block_rows, dh), lambda i: (0, i, 0)),
            pl.BlockSpec((block_rows, _L), lambda i: (i, 0)),
            pl.BlockSpec((d, h), lambda i: (0, 0)),
            pl.BlockSpec((d, h), lambda i: (0, 0)),
            pl.BlockSpec((1, h), lambda i: (0, 0)),
        ],
        out_specs=out_spec,
        out_shape=out_shape,
    )


@functools.lru_cache(maxsize=None)
def _tc_combine_stacked_x(n, d, h, relu, stacked_out, block_rows):
    dh = d // 2

    def body(x_ref, p_ref, dg_ref, ws_ref, wn_ref, b_ref, o_ref):
        inv = 1.0 / jnp.maximum(dg_ref[:, 0:1], 1.0)
        acc = jnp.dot(x_ref[0], ws_ref[0:dh, :],
                      preferred_element_type=jnp.float32)
        acc += jnp.dot(x_ref[1], ws_ref[dh:, :],
                       preferred_element_type=jnp.float32)
        acc += jnp.dot(p_ref[0] * inv, wn_ref[0:dh, :],
                       preferred_element_type=jnp.float32)
        acc += jnp.dot(p_ref[1] * inv, wn_ref[dh:, :],
                       preferred_element_type=jnp.float32)
        acc += b_ref[...]
        acc = jnp.maximum(acc, 0.0) if relu else acc
        if stacked_out:
            o_ref[0] = acc[:, : h // 2]
            o_ref[1] = acc[:, h // 2:]
        else:
            o_ref[...] = acc

    if stacked_out:
        out_spec = pl.BlockSpec((2, block_rows, h // 2), lambda i: (0, i, 0))
        out_shape = jax.ShapeDtypeStruct((2, n, h // 2), jnp.float32)
    else:
        out_spec = pl.BlockSpec((block_rows, h), lambda i: (i, 0))
        out_shape = jax.ShapeDtypeStruct((n, h), jnp.float32)

    return pl.pallas_call(
        body,
        grid=(n // block_rows,),
        in_specs=[
            pl.BlockSpec((2, block_rows, dh), lambda i: (0, i, 0)),
            pl.BlockSpec((2, block_rows, dh), lambda i: (0, i, 0)),
            pl.BlockSpec((block_rows, _L), lambda i: (i, 0)),
            pl.BlockSpec((d, h), lambda i: (0, 0)),
            pl.BlockSpec((d, h), lambda i: (0, 0)),
            pl.BlockSpec((1, h), lambda i: (0, 0)),
        ],
        out_specs=out_spec,
        out_shape=out_shape,
    )


def kernel(x, edge_index, W1_self, W1_neigh, b1, W2_self, W2_neigh, b2):
    n, d = x.shape
    e = edge_index.shape[1]
    ch = e // (_NS * _K)
    h1w = W1_self.shape[1]
    h2w = W2_self.shape[1]
    dh = d // 2

    src = edge_index[0].reshape(_NS, ch, _K)
    dst = edge_index[1].reshape(_NS, ch, _K)
    zrows = jnp.zeros((n, dh), jnp.float32)
    zdeg = jnp.zeros((n, _L), jnp.float32)
    xs = jnp.stack([x[:, :dh], x[:, dh:]])

    p1, degt = _sc_agg(n, dh, ch, True)(xs, src, dst, zrows, zdeg)
    h1 = _tc_combine(n, d, h1w, True, True, 1000)(
        x, p1, degt, W1_self, W1_neigh, b1.reshape(1, -1))
    (p2,) = _sc_agg(n, h1w // 2, ch, False)(h1, src, dst, zrows, zdeg)
    out = _tc_combine_stacked_x(n, h1w, h2w, False, False, 1000)(
        h1, p2, degt, W2_self, W2_neigh, b2.reshape(1, -1))
    return out


# speedup vs baseline: 9.9584x; 1.0907x over previous
import functools

import jax
import jax.numpy as jnp
from jax import lax
from jax.experimental import pallas as pl
from jax.experimental.pallas import tpu as pltpu
from jax.experimental.pallas import tpu_sc as plsc

_NC = 2
_NS = 16
_L = 16
_K = 100


@functools.lru_cache(maxsize=None)
def _sc_agg(n, dh, ch, with_deg):
    mesh = plsc.VectorSubcoreMesh(core_axis_name="c", subcore_axis_name="s")
    out_type = [jax.ShapeDtypeStruct((_NC, n, dh), jnp.float32)]
    if with_deg:
        out_type.append(jax.ShapeDtypeStruct((_NC, n, _L), jnp.float32))

    ns = 5
    scratch = (
        [pltpu.VMEM((ch, _K), jnp.int32)] * 2
        + [pltpu.VMEM((_K, dh), jnp.float32)] * ns
        + ([pltpu.VMEM((_K, _L), jnp.float32)] if with_deg else [])
        + [pltpu.VMEM_SHARED((n, dh), jnp.float32)]
        + ([pltpu.VMEM_SHARED((n, _L), jnp.float32)] if with_deg else [])
        + [pltpu.SemaphoreType.DMA] * (2 * ns)
    )

    stride = (n // 8 // _NS) * 8
    window = n - (_NS - 1) * stride

    def body(table, src, dst, zrows, zdeg, *rest):
        if with_deg:
            out_p, out_deg = rest[0], rest[1]
            rest = rest[2:]
        else:
            out_p = rest[0]
            rest = rest[1:]
        srcb, dstb = rest[0], rest[1]
        rows = rest[2:2 + ns]
        rest = rest[2 + ns:]
        if with_deg:
            ones, acc, dacc = rest[0], rest[1], rest[2]
            rest = rest[3:]
        else:
            ones = dacc = None
            acc = rest[0]
            rest = rest[1:]
        gsem = rest[:ns]
        ssem = rest[ns:2 * ns]

        cid = lax.axis_index("c")
        sid = lax.axis_index("s")
        half_tab = table.at[cid]

        pltpu.sync_copy(src.at[sid], srcb)
        pltpu.sync_copy(dst.at[sid], dstb)

        lo = sid * stride
        pltpu.sync_copy(zrows.at[pl.ds(lo, window)],
                        acc.at[pl.ds(lo, window)])
        if with_deg:
            pltpu.sync_copy(zdeg.at[pl.ds(lo, window)],
                            dacc.at[pl.ds(lo, window)])
            for j in range(_K):
                ones[j, :] = jnp.ones((_L,), jnp.float32)
        plsc.subcore_barrier()

        def g_start(j, ci):
            pltpu.async_copy(half_tab.at[srcb.at[ci]], rows[j], gsem[j])

        def g_wait(j, ci):
            pltpu.make_async_copy(half_tab.at[srcb.at[ci]], rows[j],
                                  gsem[j]).wait()

        def s_start(j, ci):
            pltpu.async_copy(rows[j], acc.at[dstb.at[ci]], ssem[j], add=True)

        def s_wait(j, ci):
            pltpu.make_async_copy(rows[j], acc.at[dstb.at[ci]],
                                  ssem[j]).wait()

        for j in range(ns):
            g_start(j, j)

        def loop(turn, carry):
            base = turn * ns
            for j in range(ns):
                c = base + j
                g_wait(j, c)
                s_start(j, c)
                if with_deg:
                    @pl.when(cid == lax.rem(c, 2))
                    def _():
                        pltpu.sync_copy(ones, dacc.at[dstb.at[c]], add=True)
            for j in range(ns):
                cn = base + ns + j

                @pl.when(cn < ch)
                def _():
                    s_wait(j, cn - ns)
                    g_start(j, cn)

            return carry

        lax.fori_loop(0, ch // ns, loop, 0)
        for j in range(ns):
            s_wait(j, ch - ns + j)

        plsc.subcore_barrier()
        pltpu.sync_copy(acc.at[pl.ds(lo, window)],
                        out_p.at[cid, pl.ds(lo, window)])
        if with_deg:
            pltpu.sync_copy(dacc.at[pl.ds(lo, window)],
                            out_deg.at[cid, pl.ds(lo, window)])

    return pl.kernel(body, mesh=mesh, out_type=out_type,
                     scratch_types=scratch,
                     compiler_params=pltpu.CompilerParams(
                         use_tc_tiling_on_sc=False))


@functools.lru_cache(maxsize=None)
def _tc_combine(n, d, h, relu, stacked_out, block_rows):
    dh = d // 2

    def body(x_ref, p_ref, dg_ref, ws_ref, wn_ref, b_ref, o_ref):
        deg = dg_ref[0, :, 0:1] + dg_ref[1, :, 0:1]
        inv = 1.0 / jnp.maximum(deg, 1.0)
        acc = jnp.dot(x_ref[...], ws_ref[...],
                      preferred_element_type=jnp.float32)
        acc += jnp.dot(p_ref[0] * inv, wn_ref[0:dh, :],
                       preferred_element_type=jnp.float32)
        acc += jnp.dot(p_ref[1] * inv, wn_ref[dh:, :],
                       preferred_element_type=jnp.float32)
        acc += b_ref[...]
        acc = jnp.maximum(acc, 0.0) if relu else acc
        if stacked_out:
            o_ref[0] = acc[:, : h // 2]
            o_ref[1] = acc[:, h // 2:]
        else:
            o_ref[...] = acc

    if stacked_out:
        out_spec = pl.BlockSpec((2, block_rows, h // 2), lambda i: (0, i, 0))
        out_shape = jax.ShapeDtypeStruct((2, n, h // 2), jnp.float32)
    else:
        out_spec = pl.BlockSpec((block_rows, h), lambda i: (i, 0))
        out_shape = jax.ShapeDtypeStruct((n, h), jnp.float32)

    return pl.pallas_call(
        body,
        grid=(n // block_rows,),
        in_specs=[
            pl.BlockSpec((block_rows, d), lambda i: (i, 0)),
            pl.BlockSpec((2, block_rows, dh), lambda i: (0, i, 0)),
            pl.BlockSpec((2, block_rows, _L), lambda i: (0, i, 0)),
            pl.BlockSpec((d, h), lambda i: (0, 0)),
            pl.BlockSpec((d, h), lambda i: (0, 0)),
            pl.BlockSpec((1, h), lambda i: (0, 0)),
        ],
        out_specs=out_spec,
        out_shape=out_shape,
    )


@functools.lru_cache(maxsize=None)
def _tc_combine_stacked_x(n, d, h, relu, stacked_out, block_rows):
    dh = d // 2

    def body(x_ref, p_ref, dg_ref, ws_ref, wn_ref, b_ref, o_ref):
        deg = dg_ref[0, :, 0:1] + dg_ref[1, :, 0:1]
        inv = 1.0 / jnp.maximum(deg, 1.0)
        acc = jnp.dot(x_ref[0], ws_ref[0:dh, :],
                      preferred_element_type=jnp.float32)
        acc += jnp.dot(x_ref[1], ws_ref[dh:, :],
                       preferred_element_type=jnp.float32)
        acc += jnp.dot(p_ref[0] * inv, wn_ref[0:dh, :],
                       preferred_element_type=jnp.float32)
        acc += jnp.dot(p_ref[1] * inv, wn_ref[dh:, :],
                       preferred_element_type=jnp.float32)
        acc += b_ref[...]
        acc = jnp.maximum(acc, 0.0) if relu else acc
        if stacked_out:
            o_ref[0] = acc[:, : h // 2]
            o_ref[1] = acc[:, h // 2:]
        else:
            o_ref[...] = acc

    if stacked_out:
        out_spec = pl.BlockSpec((2, block_rows, h // 2), lambda i: (0, i, 0))
        out_shape = jax.ShapeDtypeStruct((2, n, h // 2), jnp.float32)
    else:
        out_spec = pl.BlockSpec((block_rows, h), lambda i: (i, 0))
        out_shape = jax.ShapeDtypeStruct((n, h), jnp.float32)

    return pl.pallas_call(
        body,
        grid=(n // block_rows,),
        in_specs=[
            pl.BlockSpec((2, block_rows, dh), lambda i: (0, i, 0)),
            pl.BlockSpec((2, block_rows, dh), lambda i: (0, i, 0)),
            pl.BlockSpec((2, block_rows, _L), lambda i: (0, i, 0)),
            pl.BlockSpec((d, h), lambda i: (0, 0)),
            pl.BlockSpec((d, h), lambda i: (0, 0)),
            pl.BlockSpec((1, h), lambda i: (0, 0)),
        ],
        out_specs=out_spec,
        out_shape=out_shape,
    )


def kernel(x, edge_index, W1_self, W1_neigh, b1, W2_self, W2_neigh, b2):
    n, d = x.shape
    e = edge_index.shape[1]
    ch = e // (_NS * _K)
    h1w = W1_self.shape[1]
    h2w = W2_self.shape[1]
    dh = d // 2

    src = edge_index[0].reshape(_NS, ch, _K)
    dst = edge_index[1].reshape(_NS, ch, _K)
    zrows = jnp.zeros((n, dh), jnp.float32)
    zdeg = jnp.zeros((n, _L), jnp.float32)
    xs = jnp.stack([x[:, :dh], x[:, dh:]])

    p1, degt = _sc_agg(n, dh, ch, True)(xs, src, dst, zrows, zdeg)
    h1 = _tc_combine(n, d, h1w, True, True, 1000)(
        x, p1, degt, W1_self, W1_neigh, b1.reshape(1, -1))
    (p2,) = _sc_agg(n, h1w // 2, ch, False)(h1, src, dst, zrows, zdeg)
    out = _tc_combine_stacked_x(n, h1w, h2w, False, False, 1000)(
        h1, p2, degt, W2_self, W2_neigh, b2.reshape(1, -1))
    return out
